# baseline (speedup 1.0000x reference)
import functools

import numpy as np

import jax
import jax.numpy as jnp
from jax import lax
from jax.experimental import pallas as pl
from jax.experimental.pallas import tpu as pltpu
from jax.experimental.pallas import tpu_sc as plsc

NS = [45056, 11264, 2816, 704, 176]
K = 16
D_OUT = [16, 64, 128, 256]
D2 = [d // 2 for d in D_OUT]
D_IN = [8, 32, 128, 256]
GW = [16, 64, 128, 256]
BNS = [512, 512, 352, 176]


def _pad16(c):
    return ((c + 15) // 16) * 16


def _leaky(y):
    return jnp.where(y >= 0, y, 0.2 * y)


def _fold(p, pad_out=0):
    w = p["W"] * p["g"][None, :]
    b = p["b"] * p["g"] + p["beta"]
    if pad_out:
        w = jnp.pad(w, ((0, 0), (0, pad_out)))
        b = jnp.pad(b, (0, pad_out))
    return w, b


def _full(shape):
    nd = len(shape)
    return pl.BlockSpec(shape, lambda n, _nd=nd: (0,) * _nd)


def _blk(bn, *rest):
    shape = (bn,) + rest
    nd = len(shape)
    return pl.BlockSpec(shape, lambda n, _nd=nd: (n,) + (0,) * (_nd - 1))


def _kc(*gs):
    k = 1
    while any((k * g) % 128 for g in gs) and k < K:
        k *= 2
    return k


def _bdmm(x, w, nch):
    ci = x.shape[1] // nch
    if nch == 1:
        return x @ w
    return jnp.concatenate([x[:, j * ci:(j + 1) * ci] @ w
                            for j in range(nch)], axis=-1)


def _kron(wg, kc):
    return jnp.kron(jnp.eye(kc, dtype=jnp.float32), wg) if kc > 1 else wg


def _rollmax(x, group):
    m = x
    sh = group
    while sh < x.shape[1]:
        m = jnp.maximum(m, pltpu.roll(m, sh, 1))
        sh *= 2
    return m[:, 0:group]


def _pf(c):
    return 128 // c if c < 128 else 1


def _np_packL(bn, pf):
    q = bn // pf
    l = np.zeros((bn, bn), np.float32)
    for s in range(pf):
        for r in range(q):
            l[s * q + r, pf * r + s] = 1.0
    return jnp.asarray(l)


def _mm_pack(y, l_ref, pf):
    if pf == 1:
        return y
    bn = y.shape[0]
    q = bn // pf
    return jnp.concatenate(
        [l_ref[s * q:(s + 1) * q, :] @ y for s in range(pf)], axis=1)


_SC_NW = 32


@functools.lru_cache(maxsize=None)
def _make_sc_gather(d, b):
    assert b % (8 * _SC_NW) == 0 and d % 16 == 0
    rows_w = b // _SC_NW
    t = min(128, 32768 // d, rows_w)
    chunks = []
    o = 0
    while o < rows_w:
        chunks.append((o, min(t, rows_w - o)))
        o += t
    m = len(chunks)
    mesh = plsc.VectorSubcoreMesh(core_axis_name="c", subcore_axis_name="s")

    @functools.partial(
        pl.kernel, mesh=mesh,
        out_type=jax.ShapeDtypeStruct((b, d), jnp.float32),
        compiler_params=pltpu.CompilerParams(use_tc_tiling_on_sc=False),
        scratch_types=[
            pltpu.VMEM((rows_w,), jnp.int32),
            pltpu.VMEM((t, d), jnp.float32),
            pltpu.VMEM((t, d), jnp.float32),
            pltpu.SemaphoreType.DMA,
            pltpu.SemaphoreType.DMA,
        ],
    )
    def g(table_hbm, idx_hbm, out_hbm, idx_v, buf0, buf1, sem0, sem1):
        wid = lax.axis_index("s") * 2 + lax.axis_index("c")
        base = wid * rows_w
        pltpu.sync_copy(idx_hbm.at[pl.ds(base, rows_w)], idx_v)
        bufs = (buf0, buf1)
        sems = (sem0, sem1)

        def copy(off, size, p):
            return pltpu.make_async_copy(
                table_hbm.at[idx_v.at[pl.ds(off, size)]],
                bufs[p].at[pl.ds(0, size)], sems[p])

        def finish(off, size, p):
            copy(off, size, p).wait()
            pltpu.sync_copy(bufs[p].at[pl.ds(0, size)],
                            out_hbm.at[pl.ds(base + off, size)])

        if m <= 12:
            copy(chunks[0][0], chunks[0][1], 0).start()
            for ci, (off, sz) in enumerate(chunks):
                if ci + 1 < m:
                    copy(chunks[ci + 1][0], chunks[ci + 1][1],
                         (ci + 1) % 2).start()
                finish(off, sz, ci % 2)
        else:
            assert m % 2 == 0 and all(c[1] == t for c in chunks)
            copy(0, t, 0).start()

            def body(j, carry):
                o0 = 2 * j * t
                copy(o0 + t, t, 1).start()
                finish(o0, t, 0)

                @pl.when(2 * j + 2 < m)
                def _():
                    copy(o0 + 2 * t, t, 0).start()

                finish(o0 + t, t, 1)
                return carry

            lax.fori_loop(0, m // 2, body, 0)

    return g


def _sc_gather(table, idx):
    return _make_sc_gather(table.shape[1], idx.shape[0])(table, idx)


def _stage_a0(features, xyz, fc0, m1, bn):
    n = NS[0]
    d2 = D2[0]
    gw = GW[0]

    pt, px = _pf(gw), _pf(8)
    lt, lx = _np_packL(bn, pt), _np_packL(bn, px)

    def body(feat_ref, xyz_ref, fw_ref, fb_ref, mw_ref, mb_ref, lt_ref,
             lx_ref, t_ref, x_ref):
        x = _leaky(feat_ref[...] @ fw_ref[...] + fb_ref[...])
        f = _leaky(x @ mw_ref[...] + mb_ref[...])
        pad = jnp.zeros((bn, gw - d2 - 3), jnp.float32)
        t_ref[...] = _mm_pack(
            jnp.concatenate([f, xyz_ref[...], pad], axis=-1), lt_ref, pt)
        x_ref[...] = _mm_pack(x, lx_ref, px)

    fw, fb = fc0
    mw, mb = m1
    return pl.pallas_call(
        body,
        grid=(n // bn,),
        in_specs=[_blk(bn, 3), _blk(bn, 3), _full(fw.shape), _full(fb.shape),
                  _full(mw.shape), _full(mb.shape), _full(lt.shape),
                  _full(lx.shape)],
        out_specs=[_blk(bn // pt, pt * gw), _blk(bn // px, px * 8)],
        out_shape=[jax.ShapeDtypeStruct((n // pt, pt * gw), jnp.float32),
                   jax.ShapeDtypeStruct((n // px, px * 8), jnp.float32)],
    )(features, xyz, fw, fb, mw, mb, lt, lx)


def _stage_a(i, pooled, xyz, m1, bn):
    n = NS[i]
    dfi = D_IN[i]
    d2 = D2[i]
    gw = GW[i]

    pt, px = _pf(gw), _pf(dfi)
    lt, lx = _np_packL(bn, pt), _np_packL(bn, px)

    def body(p_ref, xyz_ref, mw_ref, mb_ref, lt_ref, lx_ref, t_ref, x_ref):
        x = _rollmax(p_ref[...], dfi)
        f = _leaky(x @ mw_ref[...] + mb_ref[...])
        pad = jnp.zeros((bn, gw - d2 - 3), jnp.float32)
        t_ref[...] = _mm_pack(
            jnp.concatenate([f, xyz_ref[...], pad], axis=-1), lt_ref, pt)
        x_ref[...] = _mm_pack(x, lx_ref, px)

    mw, mb = m1
    return pl.pallas_call(
        body,
        grid=(n // bn,),
        in_specs=[_blk(bn, K * dfi), _blk(bn, 3), _full(mw.shape),
                  _full(mb.shape), _full(lt.shape), _full(lx.shape)],
        out_specs=[_blk(bn // pt, pt * gw), _blk(bn // px, px * dfi)],
        out_shape=[jax.ShapeDtypeStruct((n // pt, pt * gw), jnp.float32),
                   jax.ShapeDtypeStruct((n // px, px * dfi), jnp.float32)],
    )(pooled, xyz, mw, mb, lt, lx)


def _stage_a4(pooled, d0, bn):
    n = NS[4]
    c = 512

    def body(p_ref, w_ref, b_ref, o_ref):
        x = _rollmax(p_ref[...], c)
        o_ref[...] = _leaky(x @ w_ref[...] + b_ref[...])

    w, b = d0
    return pl.pallas_call(
        body,
        grid=(n // bn,),
        in_specs=[_blk(bn, K * c), _full(w.shape), _full(b.shape)],
        out_specs=[_blk(bn, c)],
        out_shape=[jax.ShapeDtypeStruct((n, c), jnp.float32)],
    )(pooled, w, b)[0]


def _att_block(fset, attw_c, nch_l, sumq_ref):
    logits = _bdmm(fset, attw_c, nch_l)
    mg = jnp.max(logits, axis=-1, keepdims=True)
    e = jnp.exp(logits - mg)
    den = e @ sumq_ref
    agg = (fset * e) @ sumq_ref
    return agg / den


def _stage_d(i, g1, xyz, wmats, bn):
    n = NS[i]
    d2 = D2[i]
    gw = GW[i]
    d2p = _pad16(d2)
    c2 = 2 * d2
    r = K * d2
    kc_f = _kc(gw, c2, d2)
    kc_l = _kc(c2)
    kc_2 = _kc(d2)
    pg = _pf(d2p)

    lg = _np_packL(bn, pg)

    def body(g_ref, xyz_ref, seln_ref, tile_e_ref, sum3_ref, wu_ref, b1t_ref,
             pf_ref, px_ref, attw_ref, sumq_ref, amw_ref, amb_ref,
             w2_ref, b2t_ref, lg_ref, fx2_ref, fagg_ref):
        g = g_ref[...]
        xyzc = xyz_ref[...]
        neigh = g @ seln_ref[...]
        tile = xyzc @ tile_e_ref[...]
        rel = tile - neigh
        s = (rel * rel) @ sum3_ref[...]
        dist = jnp.sqrt(s + 1e-12)
        u = jnp.concatenate([dist, rel, neigh, xyzc], axis=-1)
        fxyz = _leaky(u @ wu_ref[...] + b1t_ref[...])
        fset = (_bdmm(g, pf_ref[...], K // kc_f)
                + _bdmm(fxyz, px_ref[...], K // kc_f))
        agg = _att_block(fset, attw_ref[...], K // kc_l, sumq_ref[...])
        fagg_ref[...] = _mm_pack(
            _leaky(agg @ amw_ref[...] + amb_ref[...]), lg_ref, pg)
        fx2_ref[...] = _leaky(_bdmm(fxyz, w2_ref[...], K // kc_2)
                              + b2t_ref[...])

    (seln, tile_e, sum3, wu, b1t, pf, px, attw_c, sumq, amw, amb,
     w2c, b2t) = wmats
    return pl.pallas_call(
        body,
        grid=(n // bn,),
        in_specs=[_blk(bn, K * gw), _blk(bn, 3)] + [
            _full(a.shape) for a in (seln, tile_e, sum3, wu, b1t, pf, px,
                                     attw_c, sumq, amw, amb, w2c, b2t, lg)],
        out_specs=[_blk(bn, r), _blk(bn // pg, pg * d2p)],
        out_shape=[jax.ShapeDtypeStruct((n, r), jnp.float32),
                   jax.ShapeDtypeStruct((n // pg, pg * d2p), jnp.float32)],
    )(g1, xyz, seln, tile_e, sum3, wu, b1t, pf, px, attw_c, sumq, amw, amb,
      w2c, b2t, lg)


def _stage_f(i, g2, fxyz2, x, wmats, bn):
    n = NS[i]
    d2 = D2[i]
    d2p = _pad16(d2)
    c2 = 2 * d2
    dout = D_OUT[i]
    dfi = D_IN[i]
    kc_f = _kc(d2p, c2, d2)
    kc_l = _kc(c2)
    pfx = _pf(dfi)
    lf = _np_packL(bn, pfx)

    def body(g_ref, fx_ref, x_ref, pg_ref, px_ref, attw_ref, sumq_ref,
             amw_ref, amb_ref, m2w_ref, m2b_ref, sw_ref, sb_ref, lf_ref,
             fe_ref):
        fset = (_bdmm(g_ref[...], pg_ref[...], K // kc_f)
                + _bdmm(fx_ref[...], px_ref[...], K // kc_f))
        agg = _att_block(fset, attw_ref[...], K // kc_l, sumq_ref[...])
        a = _leaky(agg @ amw_ref[...] + amb_ref[...])
        f = a @ m2w_ref[...] + m2b_ref[...]
        s = x_ref[...] @ sw_ref[...] + sb_ref[...]
        fe_ref[...] = _leaky(_mm_pack(f, lf_ref, pfx) + s)

    (pg, px, attw_c, sumq, amw, amb, m2w, m2b, sw, sb) = wmats
    return pl.pallas_call(
        body,
        grid=(n // bn,),
        in_specs=[_blk(bn, K * d2p), _blk(bn, K * d2),
                  _blk(bn // pfx, pfx * dfi)] + [
            _full(a.shape) for a in (pg, px, attw_c, sumq, amw, amb,
                                     m2w, m2b, sw, sb, lf)],
        out_specs=[_blk(bn // pfx, pfx * 2 * dout)],
        out_shape=[jax.ShapeDtypeStruct((n // pfx, pfx * 2 * dout),
                                        jnp.float32)],
    )(g2, fxyz2, x, pg, px, attw_c, sumq, amw, amb, m2w, m2b, sw, sb, lf)[0]


def _stage_dec(n, fi, skip, w_b, bn, spf=1):
    ct = fi.shape[1]
    w, b = w_b
    co = w.shape[1]
    cs = w.shape[0] - ct

    if spf == 1:
        def body(fi_ref, s_ref, w_ref, b_ref, o_ref):
            cat = jnp.concatenate([s_ref[...], fi_ref[...]], axis=-1)
            o_ref[...] = _leaky(cat @ w_ref[...] + b_ref[...])

        return pl.pallas_call(
            body,
            grid=(n // bn,),
            in_specs=[_blk(bn, ct), _blk(bn, cs), _full(w.shape), _full(b.shape)],
            out_specs=[_blk(bn, co)],
            out_shape=[jax.ShapeDtypeStruct((n, co), jnp.float32)],
        )(fi[:n], skip, w, b)[0]

    wt = jnp.kron(jnp.eye(spf, dtype=jnp.float32), w[:cs])
    wb = w[cs:]
    lp = _np_packL(bn, spf)

    def body(fi_ref, s_ref, wt_ref, wb_ref, b_ref, lp_ref, o_ref):
        y = fi_ref[...] @ wb_ref[...] + b_ref[...]
        o_ref[...] = _leaky(s_ref[...] @ wt_ref[...] + _mm_pack(y, lp_ref, spf))

    return pl.pallas_call(
        body,
        grid=(n // bn,),
        in_specs=[_blk(bn, ct), _blk(bn // spf, spf * cs), _full(wt.shape),
                  _full(wb.shape), _full(b.shape), _full(lp.shape)],
        out_specs=[_blk(bn // spf, spf * co)],
        out_shape=[jax.ShapeDtypeStruct((n // spf, spf * co), jnp.float32)],
    )(fi[:n], skip, wt, wb, b, lp)[0]


def _stage_head(fi_p, skip_p, dec3, fc1, fc2, fc, bn):
    pf = 4
    n4 = NS[0] // pf
    dw, db = dec3
    w1, b1 = fc1
    w2, b2 = fc2
    w3, b3 = fc
    eye = jnp.eye(pf, dtype=jnp.float32)
    wt = jnp.kron(eye, dw[:32])
    wb = jnp.kron(eye, dw[32:])
    dbt = jnp.tile(db, (pf,))
    w1k, b1k = jnp.kron(eye, w1), jnp.tile(b1, (pf,))
    w2k, b2k = jnp.kron(eye, w2), jnp.tile(b2, (pf,))
    w3k, b3k = jnp.kron(eye, w3), jnp.tile(b3, (pf,))

    def body(fi_ref, s_ref, wt_ref, wb_ref, db_ref, w1_ref, b1_ref, w2_ref,
             b2_ref, w3_ref, b3_ref, o_ref):
        x = _leaky(s_ref[...] @ wt_ref[...] + fi_ref[...] @ wb_ref[...]
                   + db_ref[...])
        x = _leaky(x @ w1_ref[...] + b1_ref[...])
        x = _leaky(x @ w2_ref[...] + b2_ref[...])
        o_ref[...] = x @ w3_ref[...] + b3_ref[...]

    bn4 = bn // pf
    return pl.pallas_call(
        body,
        grid=(n4 // bn4,),
        in_specs=[_blk(bn4, 128), _blk(bn4, 128)] + [
            _full(a.shape) for a in (wt, wb, dbt, w1k, b1k, w2k, b2k,
                                     w3k, b3k)],
        out_specs=[_blk(bn4, pf * 19)],
        out_shape=[jax.ShapeDtypeStruct((n4, pf * 19), jnp.float32)],
    )(fi_p, skip_p, wt, wb, dbt, w1k, b1k, w2k, b2k, w3k, b3k)[0]


def _np_seln(gw, d2):
    s = np.zeros((K * gw, K * 3), np.float32)
    for k in range(K):
        for c in range(3):
            s[k * gw + d2 + c, k * 3 + c] = 1.0
    return jnp.asarray(s)


def _np_tile_e():
    return jnp.asarray(np.tile(np.eye(3, dtype=np.float32), (1, K)))


def _np_sum3():
    return jnp.asarray(np.kron(np.eye(K, dtype=np.float32),
                               np.ones((3, 1), np.float32)))


def _np_place(gin, gout, off, d2, kc):
    p = np.zeros((gin, gout), np.float32)
    p[0:d2, off:off + d2] = np.eye(d2, dtype=np.float32)
    return jnp.asarray(np.kron(np.eye(kc, dtype=np.float32), p))


def _np_sumq(c2):
    return jnp.asarray(np.kron(np.ones((K, 1), np.float32),
                               np.eye(c2, dtype=np.float32)))


def _d_wmats(i, ep):
    d2 = D2[i]
    gw = GW[i]
    c2 = 2 * d2
    w1, b1 = _fold(ep["mlp_xyz1"])
    w2, b2 = _fold(ep["mlp_xyz2"])
    attw = ep["att1"]["attW"]
    amw, amb = _fold(ep["att1"]["mlp"], _pad16(d2) - d2)
    eye = np.eye(K, dtype=np.float32)
    wu = jnp.concatenate([
        jnp.kron(jnp.asarray(eye), w1[0:1]),
        jnp.kron(jnp.asarray(eye), w1[1:4]),
        jnp.kron(jnp.asarray(eye), w1[7:10]),
        jnp.tile(w1[4:7], (1, K)),
    ], axis=0)
    b1t = jnp.tile(b1, (K,))
    kc_f = _kc(gw, c2, d2)
    kc_l = _kc(c2)
    kc_2 = _kc(d2)
    return (
        _np_seln(gw, d2), _np_tile_e(), _np_sum3(), wu, b1t,
        _np_place(gw, c2, 0, d2, kc_f),
        _np_place(d2, c2, d2, d2, kc_f),
        _kron(attw, kc_l), _np_sumq(c2), amw, amb,
        _kron(w2, kc_2), jnp.tile(b2, (K,)),
    )


def _f_wmats(i, ep):
    d2 = D2[i]
    d2p = _pad16(d2)
    c2 = 2 * d2
    attw = ep["att2"]["attW"]
    amw, amb = _fold(ep["att2"]["mlp"])
    m2w, m2b = _fold(ep["mlp2"])
    sw, sb = _fold(ep["shortcut"])
    kc_f = _kc(d2p, c2, d2)
    kc_l = _kc(c2)
    pfx = _pf(D_IN[i])
    if pfx > 1:
        sw = jnp.kron(jnp.eye(pfx, dtype=jnp.float32), sw)
        sb = jnp.tile(sb, (pfx,))
    return (
        _np_place(d2p, c2, 0, d2, kc_f),
        _np_place(d2, c2, d2, d2, kc_f),
        _kron(attw, kc_l), _np_sumq(c2), amw, amb, m2w, m2b, sw, sb,
    )


def kernel(features, xyz_0, xyz_1, xyz_2, xyz_3, neigh_idx_0, neigh_idx_1,
           neigh_idx_2, neigh_idx_3, sub_idx_0, sub_idx_1, sub_idx_2,
           sub_idx_3, interp_idx_0, interp_idx_1, interp_idx_2, interp_idx_3,
           params):
    xyzs = [xyz_0[0], xyz_1[0], xyz_2[0], xyz_3[0]]
    nidxs = [neigh_idx_0[0].reshape(-1), neigh_idx_1[0].reshape(-1),
             neigh_idx_2[0].reshape(-1), neigh_idx_3[0].reshape(-1)]
    sidxs = [sub_idx_0[0].reshape(-1), sub_idx_1[0].reshape(-1),
             sub_idx_2[0].reshape(-1), sub_idx_3[0].reshape(-1)]
    iidxs = [interp_idx_0[0].reshape(-1), interp_idx_1[0].reshape(-1),
             interp_idx_2[0].reshape(-1), interp_idx_3[0].reshape(-1)]

    p = params
    fc0w = p["fc0"]["W"] * p["bn0"]["g"][None, :]
    fc0b = p["fc0"]["b"] * p["bn0"]["g"] + p["bn0"]["beta"]

    fe0 = None
    skips = []
    for i in range(4):
        ep = p["enc"][i]
        d2 = D2[i]
        if i == 0:
            t, x = _stage_a0(features[0], xyzs[0], (fc0w, fc0b),
                             _fold(ep["mlp1"]), BNS[0])
        else:
            pooled = _sc_gather(
                fe_prev.reshape(NS[i - 1], 2 * D_OUT[i - 1]),
                sidxs[i - 1]).reshape(NS[i], K * 2 * D_OUT[i - 1])
            t, x = _stage_a(i, pooled, xyzs[i], _fold(ep["mlp1"]), BNS[i])
            skips.append(x)
        g1 = _sc_gather(t.reshape(NS[i], GW[i]),
                        nidxs[i]).reshape(NS[i], K * GW[i])
        fxyz2, fagg = _stage_d(i, g1, xyzs[i], _d_wmats(i, ep), BNS[i])
        g2 = _sc_gather(fagg.reshape(NS[i], _pad16(d2)),
                        nidxs[i]).reshape(NS[i], K * _pad16(d2))
        fe = _stage_f(i, g2, fxyz2, x, _f_wmats(i, ep), BNS[i])
        if i == 0:
            fe0 = fe
        fe_prev = fe

    pooled = _sc_gather(fe_prev.reshape(NS[3], 512),
                        sidxs[3]).reshape(NS[4], K * 512)
    xd = _stage_a4(pooled, _fold(p["decoder_0"]), NS[4])

    dec_bns = [704, 704, 512, 512]
    xcur = xd
    tbls = [skips[2], skips[1], skips[0]]
    for j in range(3):
        n = NS[3 - j]
        ii = iidxs[3 - j]
        if ii.shape[0] % 256:
            ii = jnp.pad(ii, (0, 256 - ii.shape[0] % 256))
        fi = _sc_gather(xcur, ii)
        xcur = _stage_dec(n, fi, tbls[j], _fold(p["dec"][j]), dec_bns[j],
                          spf=(4 if j == 2 else 1))
        if j == 2:
            xcur = xcur.reshape(n, 32)
    fi = _sc_gather(xcur, iidxs[0]).reshape(NS[0] // 4, 128)
    out = _stage_head(fi, fe0.reshape(NS[0] // 4, 128),
                      _fold(p["dec"][3]), _fold(p["fc1"]), _fold(p["fc2"]),
                      (p["fc"]["W"], p["fc"]["b"]), 512)
    out = out.reshape(NS[0] // 4, 4, 19).reshape(NS[0], 19)
    return jnp.transpose(out[None], (0, 2, 1))


# speedup vs baseline: 1.0621x; 1.0621x over previous
import functools

import numpy as np

import jax
import jax.numpy as jnp
from jax import lax
from jax.experimental import pallas as pl
from jax.experimental.pallas import tpu as pltpu
from jax.experimental.pallas import tpu_sc as plsc

NS = [45056, 11264, 2816, 704, 176]
K = 16
D_OUT = [16, 64, 128, 256]
D2 = [d // 2 for d in D_OUT]
D_IN = [8, 32, 128, 256]
GW = [16, 64, 128, 256]
BNS = [512, 512, 352, 176]


def _pad16(c):
    return ((c + 15) // 16) * 16


def _leaky(y):
    return jnp.where(y >= 0, y, 0.2 * y)


def _fold(p, pad_out=0):
    w = p["W"] * p["g"][None, :]
    b = p["b"] * p["g"] + p["beta"]
    if pad_out:
        w = jnp.pad(w, ((0, 0), (0, pad_out)))
        b = jnp.pad(b, (0, pad_out))
    return w, b


def _full(shape):
    nd = len(shape)
    return pl.BlockSpec(shape, lambda n, _nd=nd: (0,) * _nd)


def _blk(bn, *rest):
    shape = (bn,) + rest
    nd = len(shape)
    return pl.BlockSpec(shape, lambda n, _nd=nd: (n,) + (0,) * (_nd - 1))


def _kc(*gs):
    k = 1
    while any((k * g) % 128 for g in gs) and k < K:
        k *= 2
    return k


def _bdmm(x, w, nch):
    ci = x.shape[1] // nch
    if nch == 1:
        return x @ w
    return jnp.concatenate([x[:, j * ci:(j + 1) * ci] @ w
                            for j in range(nch)], axis=-1)


def _kron(wg, kc):
    return jnp.kron(jnp.eye(kc, dtype=jnp.float32), wg) if kc > 1 else wg


def _rollmax(x, group):
    m = x
    sh = group
    while sh < x.shape[1]:
        m = jnp.maximum(m, pltpu.roll(m, sh, 1))
        sh *= 2
    return m[:, 0:group]


def _pf(c):
    return 128 // c if c < 128 else 1


def _np_packL(bn, pf):
    q = bn // pf
    l = np.zeros((bn, bn), np.float32)
    for s in range(pf):
        for r in range(q):
            l[s * q + r, pf * r + s] = 1.0
    return jnp.asarray(l)


def _mm_pack(y, l_ref, pf):
    if pf == 1:
        return y
    bn = y.shape[0]
    q = bn // pf
    return jnp.concatenate(
        [l_ref[s * q:(s + 1) * q, :] @ y for s in range(pf)], axis=1)


_SC_NW = 32


@functools.lru_cache(maxsize=None)
def _make_sc_gather(d, b):
    assert b % (8 * _SC_NW) == 0 and d % 16 == 0
    rows_w = b // _SC_NW
    t = min(128, 32768 // d, rows_w)
    chunks = []
    o = 0
    while o < rows_w:
        chunks.append((o, min(t, rows_w - o)))
        o += t
    m = len(chunks)
    nb = 2
    if m > 12 and m % 4 == 0 and (4 * t * d + rows_w) * 4 <= 470 * 1024:
        nb = 4
    mesh = plsc.VectorSubcoreMesh(core_axis_name="c", subcore_axis_name="s")

    @functools.partial(
        pl.kernel, mesh=mesh,
        out_type=jax.ShapeDtypeStruct((b, d), jnp.float32),
        compiler_params=pltpu.CompilerParams(use_tc_tiling_on_sc=False),
        scratch_types=[pltpu.VMEM((rows_w,), jnp.int32)]
        + [pltpu.VMEM((t, d), jnp.float32)] * nb
        + [pltpu.SemaphoreType.DMA] * nb,
    )
    def g(table_hbm, idx_hbm, out_hbm, idx_v, *bs):
        table = table_hbm
        out = out_hbm
        bufs = bs[:nb]
        sems = bs[nb:]
        wid = lax.axis_index("s") * 2 + lax.axis_index("c")
        base = wid * rows_w
        pltpu.sync_copy(idx_hbm.at[pl.ds(base, rows_w)], idx_v)

        def copy(off, size, p):
            return pltpu.make_async_copy(
                table.at[idx_v.at[pl.ds(off, size)]],
                bufs[p].at[pl.ds(0, size)], sems[p])

        def finish(off, size, p):
            copy(off, size, p).wait()
            pltpu.sync_copy(bufs[p].at[pl.ds(0, size)],
                            out.at[pl.ds(base + off, size)])

        if m <= 12:
            copy(chunks[0][0], chunks[0][1], 0).start()
            for ci, (off, sz) in enumerate(chunks):
                if ci + 1 < m:
                    copy(chunks[ci + 1][0], chunks[ci + 1][1],
                         (ci + 1) % 2).start()
                finish(off, sz, ci % 2)
        else:
            assert m % nb == 0 and all(c[1] == t for c in chunks)
            for q in range(nb - 1):
                copy(q * t, t, q).start()

            def body(j, carry):
                c0 = nb * j
                for r in range(nb):
                    ci = c0 + r
                    nxt = ci + nb - 1
                    rn = (r + nb - 1) % nb

                    @pl.when(nxt < m)
                    def _(nxt=nxt, rn=rn):
                        copy(nxt * t, t, rn).start()

                    finish(ci * t, t, r)
                return carry

            lax.fori_loop(0, m // nb, body, 0)

    return g


def _sc_gather(table, idx, d=None, oshape=None):
    d = d if d is not None else table.shape[1]
    b = idx.shape[0]
    out = _make_sc_gather(d, b)(table.reshape(-1, d), idx)
    return out.reshape(oshape) if oshape is not None else out


def _stage_a0(features, xyz, fc0, m1, bn):
    n = NS[0]
    d2 = D2[0]
    gw = GW[0]

    pt, px = _pf(gw), _pf(8)
    lt, lx = _np_packL(bn, pt), _np_packL(bn, px)

    def body(feat_ref, xyz_ref, fw_ref, fb_ref, mw_ref, mb_ref, lt_ref,
             lx_ref, t_ref, x_ref):
        x = _leaky(feat_ref[...] @ fw_ref[...] + fb_ref[...])
        f = _leaky(x @ mw_ref[...] + mb_ref[...])
        pad = jnp.zeros((bn, gw - d2 - 3), jnp.float32)
        t_ref[...] = _mm_pack(
            jnp.concatenate([f, xyz_ref[...], pad], axis=-1), lt_ref, pt)
        x_ref[...] = _mm_pack(x, lx_ref, px)

    fw, fb = fc0
    mw, mb = m1
    return pl.pallas_call(
        body,
        grid=(n // bn,),
        in_specs=[_blk(bn, 3), _blk(bn, 3), _full(fw.shape), _full(fb.shape),
                  _full(mw.shape), _full(mb.shape), _full(lt.shape),
                  _full(lx.shape)],
        out_specs=[_blk(bn // pt, pt * gw), _blk(bn // px, px * 8)],
        out_shape=[jax.ShapeDtypeStruct((n // pt, pt * gw), jnp.float32),
                   jax.ShapeDtypeStruct((n // px, px * 8), jnp.float32)],
    )(features, xyz, fw, fb, mw, mb, lt, lx)


def _stage_a(i, pooled, xyz, m1, bn):
    n = NS[i]
    dfi = D_IN[i]
    d2 = D2[i]
    gw = GW[i]

    pt, px = _pf(gw), _pf(dfi)
    lt, lx = _np_packL(bn, pt), _np_packL(bn, px)

    def body(p_ref, xyz_ref, mw_ref, mb_ref, lt_ref, lx_ref, t_ref, x_ref):
        x = _rollmax(p_ref[...], dfi)
        f = _leaky(x @ mw_ref[...] + mb_ref[...])
        pad = jnp.zeros((bn, gw - d2 - 3), jnp.float32)
        t_ref[...] = _mm_pack(
            jnp.concatenate([f, xyz_ref[...], pad], axis=-1), lt_ref, pt)
        x_ref[...] = _mm_pack(x, lx_ref, px)

    mw, mb = m1
    return pl.pallas_call(
        body,
        grid=(n // bn,),
        in_specs=[_blk(bn, K * dfi), _blk(bn, 3), _full(mw.shape),
                  _full(mb.shape), _full(lt.shape), _full(lx.shape)],
        out_specs=[_blk(bn // pt, pt * gw), _blk(bn // px, px * dfi)],
        out_shape=[jax.ShapeDtypeStruct((n // pt, pt * gw), jnp.float32),
                   jax.ShapeDtypeStruct((n // px, px * dfi), jnp.float32)],
    )(pooled, xyz, mw, mb, lt, lx)


def _stage_a4(pooled, d0, bn):
    n = NS[4]
    c = 512

    def body(p_ref, w_ref, b_ref, o_ref):
        x = _rollmax(p_ref[...], c)
        o_ref[...] = _leaky(x @ w_ref[...] + b_ref[...])

    w, b = d0
    return pl.pallas_call(
        body,
        grid=(n // bn,),
        in_specs=[_blk(bn, K * c), _full(w.shape), _full(b.shape)],
        out_specs=[_blk(bn, c)],
        out_shape=[jax.ShapeDtypeStruct((n, c), jnp.float32)],
    )(pooled, w, b)[0]


def _att_block(fset, attw_c, nch_l, sumq_ref):
    logits = _bdmm(fset, attw_c, nch_l)
    mg = jnp.max(logits, axis=-1, keepdims=True)
    e = jnp.exp(logits - mg)
    den = e @ sumq_ref
    agg = (fset * e) @ sumq_ref
    return agg / den


def _stage_d(i, g1, xyz, wmats, bn):
    n = NS[i]
    d2 = D2[i]
    gw = GW[i]
    d2p = _pad16(d2)
    c2 = 2 * d2
    r = K * d2
    kc_f = _kc(gw, c2, d2)
    kc_l = _kc(c2)
    kc_2 = _kc(d2)
    pg = _pf(d2p)

    lg = _np_packL(bn, pg)

    def body(g_ref, xyz_ref, seln_ref, tile_e_ref, sum3_ref, wu_ref, b1t_ref,
             pf_ref, px_ref, attw_ref, sumq_ref, amw_ref, amb_ref,
             w2_ref, b2t_ref, lg_ref, fx2_ref, fagg_ref):
        g = g_ref[...]
        xyzc = xyz_ref[...]
        neigh = g @ seln_ref[...]
        tile = xyzc @ tile_e_ref[...]
        rel = tile - neigh
        s = (rel * rel) @ sum3_ref[...]
        dist = jnp.sqrt(s + 1e-12)
        u = jnp.concatenate([dist, rel, neigh, xyzc], axis=-1)
        fxyz = _leaky(u @ wu_ref[...] + b1t_ref[...])
        fset = (_bdmm(g, pf_ref[...], K // kc_f)
                + _bdmm(fxyz, px_ref[...], K // kc_f))
        agg = _att_block(fset, attw_ref[...], K // kc_l, sumq_ref[...])
        fagg_ref[...] = _mm_pack(
            _leaky(agg @ amw_ref[...] + amb_ref[...]), lg_ref, pg)
        fx2_ref[...] = _leaky(_bdmm(fxyz, w2_ref[...], K // kc_2)
                              + b2t_ref[...])

    (seln, tile_e, sum3, wu, b1t, pf, px, attw_c, sumq, amw, amb,
     w2c, b2t) = wmats
    return pl.pallas_call(
        body,
        grid=(n // bn,),
        in_specs=[_blk(bn, K * gw), _blk(bn, 3)] + [
            _full(a.shape) for a in (seln, tile_e, sum3, wu, b1t, pf, px,
                                     attw_c, sumq, amw, amb, w2c, b2t, lg)],
        out_specs=[_blk(bn, r), _blk(bn // pg, pg * d2p)],
        out_shape=[jax.ShapeDtypeStruct((n, r), jnp.float32),
                   jax.ShapeDtypeStruct((n // pg, pg * d2p), jnp.float32)],
    )(g1, xyz, seln, tile_e, sum3, wu, b1t, pf, px, attw_c, sumq, amw, amb,
      w2c, b2t, lg)


def _stage_f(i, g2, fxyz2, x, wmats, bn):
    n = NS[i]
    d2 = D2[i]
    d2p = _pad16(d2)
    c2 = 2 * d2
    dout = D_OUT[i]
    dfi = D_IN[i]
    kc_f = _kc(d2p, c2, d2)
    kc_l = _kc(c2)
    pfx = _pf(dfi)
    lf = _np_packL(bn, pfx)

    def body(g_ref, fx_ref, x_ref, pg_ref, px_ref, attw_ref, sumq_ref,
             amw_ref, amb_ref, m2w_ref, m2b_ref, sw_ref, sb_ref, lf_ref,
             fe_ref):
        fset = (_bdmm(g_ref[...], pg_ref[...], K // kc_f)
                + _bdmm(fx_ref[...], px_ref[...], K // kc_f))
        agg = _att_block(fset, attw_ref[...], K // kc_l, sumq_ref[...])
        a = _leaky(agg @ amw_ref[...] + amb_ref[...])
        f = a @ m2w_ref[...] + m2b_ref[...]
        s = x_ref[...] @ sw_ref[...] + sb_ref[...]
        fe_ref[...] = _leaky(_mm_pack(f, lf_ref, pfx) + s)

    (pg, px, attw_c, sumq, amw, amb, m2w, m2b, sw, sb) = wmats
    return pl.pallas_call(
        body,
        grid=(n // bn,),
        in_specs=[_blk(bn, K * d2p), _blk(bn, K * d2),
                  _blk(bn // pfx, pfx * dfi)] + [
            _full(a.shape) for a in (pg, px, attw_c, sumq, amw, amb,
                                     m2w, m2b, sw, sb, lf)],
        out_specs=[_blk(bn // pfx, pfx * 2 * dout)],
        out_shape=[jax.ShapeDtypeStruct((n // pfx, pfx * 2 * dout),
                                        jnp.float32)],
    )(g2, fxyz2, x, pg, px, attw_c, sumq, amw, amb, m2w, m2b, sw, sb, lf)[0]


def _stage_dec(n, fi, skip, w_b, bn, spf=1):
    ct = fi.shape[1]
    w, b = w_b
    co = w.shape[1]
    cs = w.shape[0] - ct

    if spf == 1:
        def body(fi_ref, s_ref, w_ref, b_ref, o_ref):
            cat = jnp.concatenate([s_ref[...], fi_ref[...]], axis=-1)
            o_ref[...] = _leaky(cat @ w_ref[...] + b_ref[...])

        return pl.pallas_call(
            body,
            grid=(n // bn,),
            in_specs=[_blk(bn, ct), _blk(bn, cs), _full(w.shape), _full(b.shape)],
            out_specs=[_blk(bn, co)],
            out_shape=[jax.ShapeDtypeStruct((n, co), jnp.float32)],
        )(fi[:n], skip, w, b)[0]

    wt = jnp.kron(jnp.eye(spf, dtype=jnp.float32), w[:cs])
    wb = w[cs:]
    lp = _np_packL(bn, spf)

    def body(fi_ref, s_ref, wt_ref, wb_ref, b_ref, lp_ref, o_ref):
        y = fi_ref[...] @ wb_ref[...] + b_ref[...]
        o_ref[...] = _leaky(s_ref[...] @ wt_ref[...] + _mm_pack(y, lp_ref, spf))

    return pl.pallas_call(
        body,
        grid=(n // bn,),
        in_specs=[_blk(bn, ct), _blk(bn // spf, spf * cs), _full(wt.shape),
                  _full(wb.shape), _full(b.shape), _full(lp.shape)],
        out_specs=[_blk(bn // spf, spf * co)],
        out_shape=[jax.ShapeDtypeStruct((n // spf, spf * co), jnp.float32)],
    )(fi[:n], skip, wt, wb, b, lp)[0]


def _stage_head(fi_p, skip_p, dec3, fc1, fc2, fc, bn):
    pf = 4
    n4 = NS[0] // pf
    dw, db = dec3
    w1, b1 = fc1
    w2, b2 = fc2
    w3, b3 = fc
    eye = jnp.eye(pf, dtype=jnp.float32)
    wt = jnp.kron(eye, dw[:32])
    wb = jnp.kron(eye, dw[32:])
    dbt = jnp.tile(db, (pf,))
    w1k, b1k = jnp.kron(eye, w1), jnp.tile(b1, (pf,))
    w2k, b2k = jnp.kron(eye, w2), jnp.tile(b2, (pf,))
    w3k, b3k = jnp.kron(eye, w3), jnp.tile(b3, (pf,))

    def body(fi_ref, s_ref, wt_ref, wb_ref, db_ref, w1_ref, b1_ref, w2_ref,
             b2_ref, w3_ref, b3_ref, o_ref):
        x = _leaky(s_ref[...] @ wt_ref[...] + fi_ref[...] @ wb_ref[...]
                   + db_ref[...])
        x = _leaky(x @ w1_ref[...] + b1_ref[...])
        x = _leaky(x @ w2_ref[...] + b2_ref[...])
        o_ref[...] = x @ w3_ref[...] + b3_ref[...]

    bn4 = bn // pf
    return pl.pallas_call(
        body,
        grid=(n4 // bn4,),
        in_specs=[_blk(bn4, 128), _blk(bn4, 128)] + [
            _full(a.shape) for a in (wt, wb, dbt, w1k, b1k, w2k, b2k,
                                     w3k, b3k)],
        out_specs=[_blk(bn4, pf * 19)],
        out_shape=[jax.ShapeDtypeStruct((n4, pf * 19), jnp.float32)],
    )(fi_p, skip_p, wt, wb, dbt, w1k, b1k, w2k, b2k, w3k, b3k)[0]


def _np_seln(gw, d2):
    s = np.zeros((K * gw, K * 3), np.float32)
    for k in range(K):
        for c in range(3):
            s[k * gw + d2 + c, k * 3 + c] = 1.0
    return jnp.asarray(s)


def _np_tile_e():
    return jnp.asarray(np.tile(np.eye(3, dtype=np.float32), (1, K)))


def _np_sum3():
    return jnp.asarray(np.kron(np.eye(K, dtype=np.float32),
                               np.ones((3, 1), np.float32)))


def _np_place(gin, gout, off, d2, kc):
    p = np.zeros((gin, gout), np.float32)
    p[0:d2, off:off + d2] = np.eye(d2, dtype=np.float32)
    return jnp.asarray(np.kron(np.eye(kc, dtype=np.float32), p))


def _np_sumq(c2):
    return jnp.asarray(np.kron(np.ones((K, 1), np.float32),
                               np.eye(c2, dtype=np.float32)))


def _d_wmats(i, ep):
    d2 = D2[i]
    gw = GW[i]
    c2 = 2 * d2
    w1, b1 = _fold(ep["mlp_xyz1"])
    w2, b2 = _fold(ep["mlp_xyz2"])
    attw = ep["att1"]["attW"]
    amw, amb = _fold(ep["att1"]["mlp"], _pad16(d2) - d2)
    eye = np.eye(K, dtype=np.float32)
    wu = jnp.concatenate([
        jnp.kron(jnp.asarray(eye), w1[0:1]),
        jnp.kron(jnp.asarray(eye), w1[1:4]),
        jnp.kron(jnp.asarray(eye), w1[7:10]),
        jnp.tile(w1[4:7], (1, K)),
    ], axis=0)
    b1t = jnp.tile(b1, (K,))
    kc_f = _kc(gw, c2, d2)
    kc_l = _kc(c2)
    kc_2 = _kc(d2)
    return (
        _np_seln(gw, d2), _np_tile_e(), _np_sum3(), wu, b1t,
        _np_place(gw, c2, 0, d2, kc_f),
        _np_place(d2, c2, d2, d2, kc_f),
        _kron(attw, kc_l), _np_sumq(c2), amw, amb,
        _kron(w2, kc_2), jnp.tile(b2, (K,)),
    )


def _f_wmats(i, ep):
    d2 = D2[i]
    d2p = _pad16(d2)
    c2 = 2 * d2
    attw = ep["att2"]["attW"]
    amw, amb = _fold(ep["att2"]["mlp"])
    m2w, m2b = _fold(ep["mlp2"])
    sw, sb = _fold(ep["shortcut"])
    kc_f = _kc(d2p, c2, d2)
    kc_l = _kc(c2)
    pfx = _pf(D_IN[i])
    if pfx > 1:
        sw = jnp.kron(jnp.eye(pfx, dtype=jnp.float32), sw)
        sb = jnp.tile(sb, (pfx,))
    return (
        _np_place(d2p, c2, 0, d2, kc_f),
        _np_place(d2, c2, d2, d2, kc_f),
        _kron(attw, kc_l), _np_sumq(c2), amw, amb, m2w, m2b, sw, sb,
    )


def kernel(features, xyz_0, xyz_1, xyz_2, xyz_3, neigh_idx_0, neigh_idx_1,
           neigh_idx_2, neigh_idx_3, sub_idx_0, sub_idx_1, sub_idx_2,
           sub_idx_3, interp_idx_0, interp_idx_1, interp_idx_2, interp_idx_3,
           params):
    xyzs = [xyz_0[0], xyz_1[0], xyz_2[0], xyz_3[0]]
    nidxs = [neigh_idx_0[0].reshape(-1), neigh_idx_1[0].reshape(-1),
             neigh_idx_2[0].reshape(-1), neigh_idx_3[0].reshape(-1)]
    sidxs = [sub_idx_0[0].reshape(-1), sub_idx_1[0].reshape(-1),
             sub_idx_2[0].reshape(-1), sub_idx_3[0].reshape(-1)]
    iidxs = [interp_idx_0[0].reshape(-1), interp_idx_1[0].reshape(-1),
             interp_idx_2[0].reshape(-1), interp_idx_3[0].reshape(-1)]

    p = params
    fc0w = p["fc0"]["W"] * p["bn0"]["g"][None, :]
    fc0b = p["fc0"]["b"] * p["bn0"]["g"] + p["bn0"]["beta"]

    fe0 = None
    skips = []
    for i in range(4):
        ep = p["enc"][i]
        d2 = D2[i]
        if i == 0:
            t, x = _stage_a0(features[0], xyzs[0], (fc0w, fc0b),
                             _fold(ep["mlp1"]), BNS[0])
        else:
            pooled = _sc_gather(fe_prev, sidxs[i - 1], d=2 * D_OUT[i - 1],
                                oshape=(NS[i], K * 2 * D_OUT[i - 1]))
            t, x = _stage_a(i, pooled, xyzs[i], _fold(ep["mlp1"]), BNS[i])
            skips.append(x)
        g1 = _sc_gather(t, nidxs[i], d=GW[i], oshape=(NS[i], K * GW[i]))
        fxyz2, fagg = _stage_d(i, g1, xyzs[i], _d_wmats(i, ep), BNS[i])
        g2 = _sc_gather(fagg, nidxs[i], d=_pad16(d2),
                        oshape=(NS[i], K * _pad16(d2)))
        fe = _stage_f(i, g2, fxyz2, x, _f_wmats(i, ep), BNS[i])
        if i == 0:
            fe0 = fe
        fe_prev = fe

    pooled = _sc_gather(fe_prev, sidxs[3], d=512, oshape=(NS[4], K * 512))
    xd = _stage_a4(pooled, _fold(p["decoder_0"]), NS[4])

    dec_bns = [704, 704, 512, 512]
    xcur = xd
    tbls = [skips[2], skips[1], skips[0]]
    for j in range(3):
        n = NS[3 - j]
        ii = iidxs[3 - j]
        if ii.shape[0] % 256:
            ii = jnp.pad(ii, (0, 256 - ii.shape[0] % 256))
        fi = _sc_gather(xcur, ii)
        xcur = _stage_dec(n, fi, tbls[j], _fold(p["dec"][j]), dec_bns[j],
                          spf=(4 if j == 2 else 1))
    fi = _sc_gather(xcur, iidxs[0], d=32, oshape=(NS[0] // 4, 128))
    out = _stage_head(fi, fe0.reshape(NS[0] // 4, 128),
                      _fold(p["dec"][3]), _fold(p["fc1"]), _fold(p["fc2"]),
                      (p["fc"]["W"], p["fc"]["b"]), 512)
    out = out.reshape(NS[0] // 4, 4, 19).reshape(NS[0], 19)
    return jnp.transpose(out[None], (0, 2, 1))


# speedup vs baseline: 1.1208x; 1.0552x over previous
import functools

import numpy as np

import jax
import jax.numpy as jnp
from jax import lax
from jax.experimental import pallas as pl
from jax.experimental.pallas import tpu as pltpu
from jax.experimental.pallas import tpu_sc as plsc

NS = [45056, 11264, 2816, 704, 176]
K = 16
D_OUT = [16, 64, 128, 256]
D2 = [d // 2 for d in D_OUT]
D_IN = [8, 32, 128, 256]
GW = [16, 64, 128, 256]
BNS = [1024, 1024, 704, 352]


def _pad16(c):
    return ((c + 15) // 16) * 16


def _leaky(y):
    return jnp.where(y >= 0, y, 0.2 * y)


def _fold(p, pad_out=0):
    w = p["W"] * p["g"][None, :]
    b = p["b"] * p["g"] + p["beta"]
    if pad_out:
        w = jnp.pad(w, ((0, 0), (0, pad_out)))
        b = jnp.pad(b, (0, pad_out))
    return w, b


def _full(shape):
    nd = len(shape)
    return pl.BlockSpec(shape, lambda n, _nd=nd: (0,) * _nd)


def _blk(bn, *rest):
    shape = (bn,) + rest
    nd = len(shape)
    return pl.BlockSpec(shape, lambda n, _nd=nd: (n,) + (0,) * (_nd - 1))


def _kc(*gs):
    k = 1
    while any((k * g) % 128 for g in gs) and k < K:
        k *= 2
    return k


def _bdmm(x, w, nch):
    ci = x.shape[1] // nch
    if nch == 1:
        return x @ w
    return jnp.concatenate([x[:, j * ci:(j + 1) * ci] @ w
                            for j in range(nch)], axis=-1)


def _kron(wg, kc):
    return jnp.kron(jnp.eye(kc, dtype=jnp.float32), wg) if kc > 1 else wg


def _rollmax(x, group):
    m = x
    sh = group
    while sh < x.shape[1]:
        m = jnp.maximum(m, pltpu.roll(m, sh, 1))
        sh *= 2
    return m[:, 0:group]


def _pf(c):
    return 128 // c if c < 128 else 1


def _np_packL(bn, pf):
    q = bn // pf
    l = np.zeros((bn, bn), np.float32)
    for s in range(pf):
        for r in range(q):
            l[s * q + r, pf * r + s] = 1.0
    return jnp.asarray(l)


def _mm_pack(y, l_ref, pf):
    if pf == 1:
        return y
    bn = y.shape[0]
    q = bn // pf
    return jnp.concatenate(
        [l_ref[s * q:(s + 1) * q, :] @ y for s in range(pf)], axis=1)


_SC_NW = 32


@functools.lru_cache(maxsize=None)
def _make_sc_gather(d, b):
    assert b % (8 * _SC_NW) == 0 and d % 16 == 0
    rows_w = b // _SC_NW
    t = min(128, 32768 // d, rows_w)
    chunks = []
    o = 0
    while o < rows_w:
        chunks.append((o, min(t, rows_w - o)))
        o += t
    m = len(chunks)
    nb = 2
    if m > 12 and m % 4 == 0 and (4 * t * d + rows_w) * 4 <= 470 * 1024:
        nb = 4
    mesh = plsc.VectorSubcoreMesh(core_axis_name="c", subcore_axis_name="s")

    @functools.partial(
        pl.kernel, mesh=mesh,
        out_type=jax.ShapeDtypeStruct((b, d), jnp.float32),
        compiler_params=pltpu.CompilerParams(use_tc_tiling_on_sc=False),
        scratch_types=[pltpu.VMEM((rows_w,), jnp.int32)]
        + [pltpu.VMEM((t, d), jnp.float32)] * nb
        + [pltpu.SemaphoreType.DMA] * nb,
    )
    def g(table_hbm, idx_hbm, out_hbm, idx_v, *bs):
        table = table_hbm
        out = out_hbm
        bufs = bs[:nb]
        sems = bs[nb:]
        wid = lax.axis_index("s") * 2 + lax.axis_index("c")
        base = wid * rows_w
        pltpu.sync_copy(idx_hbm.at[pl.ds(base, rows_w)], idx_v)

        def copy(off, size, p):
            return pltpu.make_async_copy(
                table.at[idx_v.at[pl.ds(off, size)]],
                bufs[p].at[pl.ds(0, size)], sems[p])

        def finish(off, size, p):
            copy(off, size, p).wait()
            pltpu.sync_copy(bufs[p].at[pl.ds(0, size)],
                            out.at[pl.ds(base + off, size)])

        if m <= 12:
            copy(chunks[0][0], chunks[0][1], 0).start()
            for ci, (off, sz) in enumerate(chunks):
                if ci + 1 < m:
                    copy(chunks[ci + 1][0], chunks[ci + 1][1],
                         (ci + 1) % 2).start()
                finish(off, sz, ci % 2)
        else:
            assert m % nb == 0 and all(c[1] == t for c in chunks)
            for q in range(nb - 1):
                copy(q * t, t, q).start()

            def body(j, carry):
                c0 = nb * j
                for r in range(nb):
                    ci = c0 + r
                    nxt = ci + nb - 1
                    rn = (r + nb - 1) % nb

                    @pl.when(nxt < m)
                    def _(nxt=nxt, rn=rn):
                        copy(nxt * t, t, rn).start()

                    finish(ci * t, t, r)
                return carry

            lax.fori_loop(0, m // nb, body, 0)

    return g


def _sc_gather(table, idx, d=None, oshape=None):
    d = d if d is not None else table.shape[1]
    b = idx.shape[0]
    out = _make_sc_gather(d, b)(table.reshape(-1, d), idx)
    return out.reshape(oshape) if oshape is not None else out


def _stage_a0(features, xyz, fc0, m1, bn):
    n = NS[0]
    d2 = D2[0]
    gw = GW[0]

    pt, px = _pf(gw), _pf(8)
    lt, lx = _np_packL(bn, pt), _np_packL(bn, px)

    def body(feat_ref, xyz_ref, fw_ref, fb_ref, mw_ref, mb_ref, lt_ref,
             lx_ref, t_ref, x_ref):
        x = _leaky(feat_ref[...] @ fw_ref[...] + fb_ref[...])
        f = _leaky(x @ mw_ref[...] + mb_ref[...])
        pad = jnp.zeros((bn, gw - d2 - 3), jnp.float32)
        t_ref[...] = _mm_pack(
            jnp.concatenate([f, xyz_ref[...], pad], axis=-1), lt_ref, pt)
        x_ref[...] = _mm_pack(x, lx_ref, px)

    fw, fb = fc0
    mw, mb = m1
    return pl.pallas_call(
        body,
        grid=(n // bn,),
        in_specs=[_blk(bn, 3), _blk(bn, 3), _full(fw.shape), _full(fb.shape),
                  _full(mw.shape), _full(mb.shape), _full(lt.shape),
                  _full(lx.shape)],
        out_specs=[_blk(bn // pt, pt * gw), _blk(bn // px, px * 8)],
        out_shape=[jax.ShapeDtypeStruct((n // pt, pt * gw), jnp.float32),
                   jax.ShapeDtypeStruct((n // px, px * 8), jnp.float32)],
    )(features, xyz, fw, fb, mw, mb, lt, lx)


def _stage_a(i, pooled, xyz, m1, bn):
    n = NS[i]
    dfi = D_IN[i]
    d2 = D2[i]
    gw = GW[i]

    pt, px = _pf(gw), _pf(dfi)
    lt, lx = _np_packL(bn, pt), _np_packL(bn, px)

    def body(p_ref, xyz_ref, mw_ref, mb_ref, lt_ref, lx_ref, t_ref, x_ref):
        x = _rollmax(p_ref[...], dfi)
        f = _leaky(x @ mw_ref[...] + mb_ref[...])
        pad = jnp.zeros((bn, gw - d2 - 3), jnp.float32)
        t_ref[...] = _mm_pack(
            jnp.concatenate([f, xyz_ref[...], pad], axis=-1), lt_ref, pt)
        x_ref[...] = _mm_pack(x, lx_ref, px)

    mw, mb = m1
    return pl.pallas_call(
        body,
        grid=(n // bn,),
        in_specs=[_blk(bn, K * dfi), _blk(bn, 3), _full(mw.shape),
                  _full(mb.shape), _full(lt.shape), _full(lx.shape)],
        out_specs=[_blk(bn // pt, pt * gw), _blk(bn // px, px * dfi)],
        out_shape=[jax.ShapeDtypeStruct((n // pt, pt * gw), jnp.float32),
                   jax.ShapeDtypeStruct((n // px, px * dfi), jnp.float32)],
    )(pooled, xyz, mw, mb, lt, lx)


def _stage_a4(pooled, d0, bn):
    n = NS[4]
    c = 512

    def body(p_ref, w_ref, b_ref, o_ref):
        x = _rollmax(p_ref[...], c)
        o_ref[...] = _leaky(x @ w_ref[...] + b_ref[...])

    w, b = d0
    return pl.pallas_call(
        body,
        grid=(n // bn,),
        in_specs=[_blk(bn, K * c), _full(w.shape), _full(b.shape)],
        out_specs=[_blk(bn, c)],
        out_shape=[jax.ShapeDtypeStruct((n, c), jnp.float32)],
    )(pooled, w, b)[0]


def _att_block(fset, attw_c, nch_l, sumq_ref):
    logits = _bdmm(fset, attw_c, nch_l)
    mg = jnp.max(logits, axis=-1, keepdims=True)
    e = jnp.exp(logits - mg)
    den = e @ sumq_ref
    agg = (fset * e) @ sumq_ref
    return agg / den


def _stage_d(i, g1, xyz, wmats, bn):
    n = NS[i]
    d2 = D2[i]
    gw = GW[i]
    d2p = _pad16(d2)
    c2 = 2 * d2
    r = K * d2
    kc_f = _kc(gw, c2, d2)
    kc_l = _kc(c2)
    kc_2 = _kc(d2)
    pg = _pf(d2p)

    lg = _np_packL(bn, pg)

    def body(g_ref, xyz_ref, seln_ref, tile_e_ref, sum3_ref, wu_ref, b1t_ref,
             pf_ref, px_ref, attw_ref, sumq_ref, amw_ref, amb_ref,
             w2_ref, b2t_ref, lg_ref, fx2_ref, fagg_ref):
        g = g_ref[...]
        xyzc = xyz_ref[...]
        neigh = g @ seln_ref[...]
        tile = xyzc @ tile_e_ref[...]
        rel = tile - neigh
        s = (rel * rel) @ sum3_ref[...]
        dist = jnp.sqrt(s + 1e-12)
        u = jnp.concatenate([dist, rel, neigh, xyzc], axis=-1)
        fxyz = _leaky(u @ wu_ref[...] + b1t_ref[...])
        fset = (_bdmm(g, pf_ref[...], K // kc_f)
                + _bdmm(fxyz, px_ref[...], K // kc_f))
        agg = _att_block(fset, attw_ref[...], K // kc_l, sumq_ref[...])
        fagg_ref[...] = _mm_pack(
            _leaky(agg @ amw_ref[...] + amb_ref[...]), lg_ref, pg)
        fx2_ref[...] = _leaky(_bdmm(fxyz, w2_ref[...], K // kc_2)
                              + b2t_ref[...])

    (seln, tile_e, sum3, wu, b1t, pf, px, attw_c, sumq, amw, amb,
     w2c, b2t) = wmats
    return pl.pallas_call(
        body,
        grid=(n // bn,),
        in_specs=[_blk(bn, K * gw), _blk(bn, 3)] + [
            _full(a.shape) for a in (seln, tile_e, sum3, wu, b1t, pf, px,
                                     attw_c, sumq, amw, amb, w2c, b2t, lg)],
        out_specs=[_blk(bn, r), _blk(bn // pg, pg * d2p)],
        out_shape=[jax.ShapeDtypeStruct((n, r), jnp.float32),
                   jax.ShapeDtypeStruct((n // pg, pg * d2p), jnp.float32)],
    )(g1, xyz, seln, tile_e, sum3, wu, b1t, pf, px, attw_c, sumq, amw, amb,
      w2c, b2t, lg)


def _stage_f(i, g2, fxyz2, x, wmats, bn):
    n = NS[i]
    d2 = D2[i]
    d2p = _pad16(d2)
    c2 = 2 * d2
    dout = D_OUT[i]
    dfi = D_IN[i]
    kc_f = _kc(d2p, c2, d2)
    kc_l = _kc(c2)
    pfx = _pf(dfi)
    lf = _np_packL(bn, pfx)

    def body(g_ref, fx_ref, x_ref, pg_ref, px_ref, attw_ref, sumq_ref,
             amw_ref, amb_ref, m2w_ref, m2b_ref, sw_ref, sb_ref, lf_ref,
             fe_ref):
        fset = (_bdmm(g_ref[...], pg_ref[...], K // kc_f)
                + _bdmm(fx_ref[...], px_ref[...], K // kc_f))
        agg = _att_block(fset, attw_ref[...], K // kc_l, sumq_ref[...])
        a = _leaky(agg @ amw_ref[...] + amb_ref[...])
        f = a @ m2w_ref[...] + m2b_ref[...]
        s = x_ref[...] @ sw_ref[...] + sb_ref[...]
        fe_ref[...] = _leaky(_mm_pack(f, lf_ref, pfx) + s)

    (pg, px, attw_c, sumq, amw, amb, m2w, m2b, sw, sb) = wmats
    return pl.pallas_call(
        body,
        grid=(n // bn,),
        in_specs=[_blk(bn, K * d2p), _blk(bn, K * d2),
                  _blk(bn // pfx, pfx * dfi)] + [
            _full(a.shape) for a in (pg, px, attw_c, sumq, amw, amb,
                                     m2w, m2b, sw, sb, lf)],
        out_specs=[_blk(bn // pfx, pfx * 2 * dout)],
        out_shape=[jax.ShapeDtypeStruct((n // pfx, pfx * 2 * dout),
                                        jnp.float32)],
    )(g2, fxyz2, x, pg, px, attw_c, sumq, amw, amb, m2w, m2b, sw, sb, lf)[0]


def _stage_dec(n, fi, skip, w_b, bn, spf=1):
    ct = fi.shape[1]
    w, b = w_b
    co = w.shape[1]
    cs = w.shape[0] - ct

    if spf == 1:
        def body(fi_ref, s_ref, w_ref, b_ref, o_ref):
            cat = jnp.concatenate([s_ref[...], fi_ref[...]], axis=-1)
            o_ref[...] = _leaky(cat @ w_ref[...] + b_ref[...])

        return pl.pallas_call(
            body,
            grid=(n // bn,),
            in_specs=[_blk(bn, ct), _blk(bn, cs), _full(w.shape), _full(b.shape)],
            out_specs=[_blk(bn, co)],
            out_shape=[jax.ShapeDtypeStruct((n, co), jnp.float32)],
        )(fi[:n], skip, w, b)[0]

    wt = jnp.kron(jnp.eye(spf, dtype=jnp.float32), w[:cs])
    wb = w[cs:]
    lp = _np_packL(bn, spf)

    def body(fi_ref, s_ref, wt_ref, wb_ref, b_ref, lp_ref, o_ref):
        y = fi_ref[...] @ wb_ref[...] + b_ref[...]
        o_ref[...] = _leaky(s_ref[...] @ wt_ref[...] + _mm_pack(y, lp_ref, spf))

    return pl.pallas_call(
        body,
        grid=(n // bn,),
        in_specs=[_blk(bn, ct), _blk(bn // spf, spf * cs), _full(wt.shape),
                  _full(wb.shape), _full(b.shape), _full(lp.shape)],
        out_specs=[_blk(bn // spf, spf * co)],
        out_shape=[jax.ShapeDtypeStruct((n // spf, spf * co), jnp.float32)],
    )(fi[:n], skip, wt, wb, b, lp)[0]


def _stage_head(fi_p, skip_p, dec3, fc1, fc2, fc, bn):
    pf = 4
    n4 = NS[0] // pf
    dw, db = dec3
    w1, b1 = fc1
    w2, b2 = fc2
    w3, b3 = fc
    eye = jnp.eye(pf, dtype=jnp.float32)
    wt = jnp.kron(eye, dw[:32])
    wb = jnp.kron(eye, dw[32:])
    dbt = jnp.tile(db, (pf,))
    w1k, b1k = jnp.kron(eye, w1), jnp.tile(b1, (pf,))
    w2k, b2k = jnp.kron(eye, w2), jnp.tile(b2, (pf,))
    w3k, b3k = jnp.kron(eye, w3), jnp.tile(b3, (pf,))

    def body(fi_ref, s_ref, wt_ref, wb_ref, db_ref, w1_ref, b1_ref, w2_ref,
             b2_ref, w3_ref, b3_ref, o_ref):
        x = _leaky(s_ref[...] @ wt_ref[...] + fi_ref[...] @ wb_ref[...]
                   + db_ref[...])
        x = _leaky(x @ w1_ref[...] + b1_ref[...])
        x = _leaky(x @ w2_ref[...] + b2_ref[...])
        o_ref[...] = x @ w3_ref[...] + b3_ref[...]

    bn4 = bn // pf
    return pl.pallas_call(
        body,
        grid=(n4 // bn4,),
        in_specs=[_blk(bn4, 128), _blk(bn4, 128)] + [
            _full(a.shape) for a in (wt, wb, dbt, w1k, b1k, w2k, b2k,
                                     w3k, b3k)],
        out_specs=[_blk(bn4, pf * 19)],
        out_shape=[jax.ShapeDtypeStruct((n4, pf * 19), jnp.float32)],
    )(fi_p, skip_p, wt, wb, dbt, w1k, b1k, w2k, b2k, w3k, b3k)[0]


def _np_seln(gw, d2):
    s = np.zeros((K * gw, K * 3), np.float32)
    for k in range(K):
        for c in range(3):
            s[k * gw + d2 + c, k * 3 + c] = 1.0
    return jnp.asarray(s)


def _np_tile_e():
    return jnp.asarray(np.tile(np.eye(3, dtype=np.float32), (1, K)))


def _np_sum3():
    return jnp.asarray(np.kron(np.eye(K, dtype=np.float32),
                               np.ones((3, 1), np.float32)))


def _np_place(gin, gout, off, d2, kc):
    p = np.zeros((gin, gout), np.float32)
    p[0:d2, off:off + d2] = np.eye(d2, dtype=np.float32)
    return jnp.asarray(np.kron(np.eye(kc, dtype=np.float32), p))


def _np_sumq(c2):
    return jnp.asarray(np.kron(np.ones((K, 1), np.float32),
                               np.eye(c2, dtype=np.float32)))


def _d_wmats(i, ep):
    d2 = D2[i]
    gw = GW[i]
    c2 = 2 * d2
    w1, b1 = _fold(ep["mlp_xyz1"])
    w2, b2 = _fold(ep["mlp_xyz2"])
    attw = ep["att1"]["attW"]
    amw, amb = _fold(ep["att1"]["mlp"], _pad16(d2) - d2)
    eye = np.eye(K, dtype=np.float32)
    wu = jnp.concatenate([
        jnp.kron(jnp.asarray(eye), w1[0:1]),
        jnp.kron(jnp.asarray(eye), w1[1:4]),
        jnp.kron(jnp.asarray(eye), w1[7:10]),
        jnp.tile(w1[4:7], (1, K)),
    ], axis=0)
    b1t = jnp.tile(b1, (K,))
    kc_f = _kc(gw, c2, d2)
    kc_l = _kc(c2)
    kc_2 = _kc(d2)
    return (
        _np_seln(gw, d2), _np_tile_e(), _np_sum3(), wu, b1t,
        _np_place(gw, c2, 0, d2, kc_f),
        _np_place(d2, c2, d2, d2, kc_f),
        _kron(attw, kc_l), _np_sumq(c2), amw, amb,
        _kron(w2, kc_2), jnp.tile(b2, (K,)),
    )


def _f_wmats(i, ep):
    d2 = D2[i]
    d2p = _pad16(d2)
    c2 = 2 * d2
    attw = ep["att2"]["attW"]
    amw, amb = _fold(ep["att2"]["mlp"])
    m2w, m2b = _fold(ep["mlp2"])
    sw, sb = _fold(ep["shortcut"])
    kc_f = _kc(d2p, c2, d2)
    kc_l = _kc(c2)
    pfx = _pf(D_IN[i])
    if pfx > 1:
        sw = jnp.kron(jnp.eye(pfx, dtype=jnp.float32), sw)
        sb = jnp.tile(sb, (pfx,))
    return (
        _np_place(d2p, c2, 0, d2, kc_f),
        _np_place(d2, c2, d2, d2, kc_f),
        _kron(attw, kc_l), _np_sumq(c2), amw, amb, m2w, m2b, sw, sb,
    )


def kernel(features, xyz_0, xyz_1, xyz_2, xyz_3, neigh_idx_0, neigh_idx_1,
           neigh_idx_2, neigh_idx_3, sub_idx_0, sub_idx_1, sub_idx_2,
           sub_idx_3, interp_idx_0, interp_idx_1, interp_idx_2, interp_idx_3,
           params):
    xyzs = [xyz_0[0], xyz_1[0], xyz_2[0], xyz_3[0]]
    nidxs = [neigh_idx_0[0].reshape(-1), neigh_idx_1[0].reshape(-1),
             neigh_idx_2[0].reshape(-1), neigh_idx_3[0].reshape(-1)]
    sidxs = [sub_idx_0[0].reshape(-1), sub_idx_1[0].reshape(-1),
             sub_idx_2[0].reshape(-1), sub_idx_3[0].reshape(-1)]
    iidxs = [interp_idx_0[0].reshape(-1), interp_idx_1[0].reshape(-1),
             interp_idx_2[0].reshape(-1), interp_idx_3[0].reshape(-1)]

    p = params
    fc0w = p["fc0"]["W"] * p["bn0"]["g"][None, :]
    fc0b = p["fc0"]["b"] * p["bn0"]["g"] + p["bn0"]["beta"]

    fe0 = None
    skips = []
    for i in range(4):
        ep = p["enc"][i]
        d2 = D2[i]
        if i == 0:
            t, x = _stage_a0(features[0], xyzs[0], (fc0w, fc0b),
                             _fold(ep["mlp1"]), BNS[0])
        else:
            pooled = _sc_gather(fe_prev, sidxs[i - 1], d=2 * D_OUT[i - 1],
                                oshape=(NS[i], K * 2 * D_OUT[i - 1]))
            t, x = _stage_a(i, pooled, xyzs[i], _fold(ep["mlp1"]), BNS[i])
            skips.append(x)
        g1 = _sc_gather(t, nidxs[i], d=GW[i], oshape=(NS[i], K * GW[i]))
        fxyz2, fagg = _stage_d(i, g1, xyzs[i], _d_wmats(i, ep), BNS[i])
        g2 = _sc_gather(fagg, nidxs[i], d=_pad16(d2),
                        oshape=(NS[i], K * _pad16(d2)))
        fe = _stage_f(i, g2, fxyz2, x, _f_wmats(i, ep), BNS[i])
        if i == 0:
            fe0 = fe
        fe_prev = fe

    pooled = _sc_gather(fe_prev, sidxs[3], d=512, oshape=(NS[4], K * 512))
    xd = _stage_a4(pooled, _fold(p["decoder_0"]), NS[4])

    dec_bns = [704, 704, 512, 512]
    xcur = xd
    tbls = [skips[2], skips[1], skips[0]]
    for j in range(3):
        n = NS[3 - j]
        ii = iidxs[3 - j]
        if ii.shape[0] % 256:
            ii = jnp.pad(ii, (0, 256 - ii.shape[0] % 256))
        fi = _sc_gather(xcur, ii)
        xcur = _stage_dec(n, fi, tbls[j], _fold(p["dec"][j]), dec_bns[j],
                          spf=(4 if j == 2 else 1))
    fi = _sc_gather(xcur, iidxs[0], d=32, oshape=(NS[0] // 4, 128))
    out = _stage_head(fi, fe0.reshape(NS[0] // 4, 128),
                      _fold(p["dec"][3]), _fold(p["fc1"]), _fold(p["fc2"]),
                      (p["fc"]["W"], p["fc"]["b"]), 512)
    out = out.reshape(NS[0] // 4, 4, 19).reshape(NS[0], 19)
    return jnp.transpose(out[None], (0, 2, 1))
